# Initial kernel scaffold; baseline (speedup 1.0000x reference)
import jax
import jax.numpy as jnp
from jax.experimental import pallas as pl


def kernel(x, pos_table):
    raise NotImplementedError("write your pallas kernel here")


# speedup vs baseline: 1.4503x; 1.4503x over previous
import jax
import jax.numpy as jnp
from jax.experimental import pallas as pl

_S_BLK = 512


def _add_kernel(x_ref, t_ref, o_ref):
    o_ref[...] = x_ref[...] + t_ref[...]


def kernel(x, pos_table):
    batch, seq, d = x.shape
    s_blk = _S_BLK
    return pl.pallas_call(
        _add_kernel,
        grid=(seq // s_blk, batch),
        in_specs=[
            pl.BlockSpec((1, s_blk, d), lambda s, b: (b, s, 0)),
            pl.BlockSpec((s_blk, d), lambda s, b: (s, 0)),
        ],
        out_specs=pl.BlockSpec((1, s_blk, d), lambda s, b: (b, s, 0)),
        out_shape=jax.ShapeDtypeStruct(x.shape, x.dtype),
    )(x, pos_table)


# speedup vs baseline: 1.6782x; 1.1572x over previous
import jax
import jax.numpy as jnp
from jax.experimental import pallas as pl

_S_BLK = 1024


def _add_kernel(x_ref, t_ref, o_ref):
    o_ref[...] = x_ref[...] + t_ref[...]


def kernel(x, pos_table):
    batch, seq, d = x.shape
    s_blk = _S_BLK
    return pl.pallas_call(
        _add_kernel,
        grid=(seq // s_blk, batch),
        in_specs=[
            pl.BlockSpec((1, s_blk, d), lambda s, b: (b, s, 0)),
            pl.BlockSpec((s_blk, d), lambda s, b: (s, 0)),
        ],
        out_specs=pl.BlockSpec((1, s_blk, d), lambda s, b: (b, s, 0)),
        out_shape=jax.ShapeDtypeStruct(x.shape, x.dtype),
    )(x, pos_table)


# speedup vs baseline: 1.7961x; 1.0702x over previous
import jax
import jax.numpy as jnp
from jax.experimental import pallas as pl

_S_BLK = 2048


def _add_kernel(x_ref, t_ref, o_ref):
    o_ref[...] = x_ref[...] + t_ref[...]


def kernel(x, pos_table):
    batch, seq, d = x.shape
    s_blk = _S_BLK
    return pl.pallas_call(
        _add_kernel,
        grid=(seq // s_blk, batch),
        in_specs=[
            pl.BlockSpec((1, s_blk, d), lambda s, b: (b, s, 0)),
            pl.BlockSpec((s_blk, d), lambda s, b: (s, 0)),
        ],
        out_specs=pl.BlockSpec((1, s_blk, d), lambda s, b: (b, s, 0)),
        out_shape=jax.ShapeDtypeStruct(x.shape, x.dtype),
    )(x, pos_table)
